# baseline (speedup 1.0000x reference)
import functools

import jax
import jax.numpy as jnp
from jax import lax
from jax.experimental import pallas as pl
from jax.experimental.pallas import tpu as pltpu
from jax.experimental.pallas import tpu_sc as plsc

N = 10000
NP = 10240
E = 320000
NC = 2
NS = 16
HALF = NP // 2
QTR = NP // 4
CHUNK = 125
ZROWS = 32
NBW = 4
EPP = E // (NC * NS)
CAPP = 3000
NSLOTS = NC * NS * CAPP
WCH = NSLOTS // NS // CHUNK
PAD = QTR


def _partition(src1d, dst1d):
  @functools.partial(
      pl.kernel,
      out_type=(jax.ShapeDtypeStruct((4, NC * NS, CAPP), jnp.int32),
                jax.ShapeDtypeStruct((4, NC * NS, CAPP), jnp.int32)),
      mesh=plsc.VectorSubcoreMesh(core_axis_name="c", subcore_axis_name="s"),
      compiler_params=pltpu.CompilerParams(use_tc_tiling_on_sc=False,
                                           needs_layout_passes=False),
      scratch_types=[
          pltpu.VMEM((EPP,), jnp.int32),
          pltpu.VMEM((EPP,), jnp.int32),
      ] + [pltpu.VMEM((CAPP + 16,), jnp.int32)] * 8,
  )
  def k(src_hbm, dst_hbm, ps_hbm, pd_hbm, srcv, dstv, *stg):
    sbufs = stg[0:8:2]
    dbufs = stg[1:8:2]
    w = lax.axis_index("c") * NS + lax.axis_index("s")
    pltpu.sync_copy(src_hbm.at[pl.ds(w * EPP, EPP)], srcv)
    pltpu.sync_copy(dst_hbm.at[pl.ds(w * EPP, EPP)], dstv)

    zero4 = (jnp.int32(0),) * 4

    @pl.loop(0, EPP // 16, init_carry=zero4)
    def offs(g, carry):
      sv = srcv[pl.ds(g * 16, 16)]
      dv = dstv[pl.ds(g * 16, 16)]
      new = []
      for q in range(4):
        off = carry[q]
        m = (dv >= q * QTR) & (dv < (q + 1) * QTR)
        plsc.store_compressed(sbufs[q].at[pl.ds(off, 16)], sv, mask=m)
        plsc.store_compressed(dbufs[q].at[pl.ds(off, 16)], dv - q * QTR,
                              mask=m)
        new.append(off + jnp.sum(m.astype(jnp.int32), axis=0))
      return tuple(new)

    for q in range(4):
      off = jnp.minimum(offs[q], CAPP)

      @pl.loop(0, (CAPP - off + 15) // 16)
      def _(t, q=q, off=off):
        sbufs[q][pl.ds(off + t * 16, 16)] = (
            (t % 128) * 16 + lax.iota(jnp.int32, 16))
        dbufs[q][pl.ds(off + t * 16, 16)] = (
            PAD + (t % 4) * 16 + lax.iota(jnp.int32, 16))

      pltpu.sync_copy(sbufs[q].at[pl.ds(0, CAPP)], ps_hbm.at[q, w])
      pltpu.sync_copy(dbufs[q].at[pl.ds(0, CAPP)], pd_hbm.at[q, w])

  return k(src1d, dst1d)


def _segP(tables, psrc2d, pdst2d):
  nt = len(tables)
  D = 128
  rps = QTR // NS

  @functools.partial(
      pl.kernel,
      out_type=tuple(jax.ShapeDtypeStruct((NP, D), jnp.float32)
                     for _ in range(nt)),
      mesh=plsc.VectorSubcoreMesh(core_axis_name="c", subcore_axis_name="s"),
      compiler_params=pltpu.CompilerParams(use_tc_tiling_on_sc=False),
      scratch_types=[
          pltpu.VMEM((2 * WCH, CHUNK), jnp.int32),
          pltpu.VMEM((2 * WCH, CHUNK), jnp.int32),
      ] + [pltpu.VMEM((CHUNK, D), jnp.float32)] * NBW + [
          pltpu.VMEM((ZROWS, D), jnp.float32),
          pltpu.VMEM_SHARED((QTR + 64, D), jnp.float32),
      ] + [pltpu.SemaphoreType.DMA] * (2 * NBW),
  )
  def k(*refs):
    x_hbms = refs[:nt]
    psrc_hbm, pdst_hbm = refs[nt], refs[nt + 1]
    o_hbms = refs[nt + 2:2 * nt + 2]
    rest = refs[2 * nt + 2:]
    srcv, dstv = rest[0], rest[1]
    bufs = rest[2:2 + NBW]
    zbuf, acc = rest[2 + NBW], rest[3 + NBW]
    gsems = rest[4 + NBW:4 + 2 * NBW]
    ssems = rest[4 + 2 * NBW:4 + 3 * NBW]

    cid = lax.axis_index("c")
    sid = lax.axis_index("s")
    row0 = sid * rps

    @pl.loop(0, ZROWS)
    def _(r):
      @pl.loop(0, D, step=16)
      def _(c):
        zbuf[r, pl.ds(c, 16)] = jnp.zeros((16,), jnp.float32)

    cpb = NSLOTS // CHUNK
    for hb in range(2):
      base = (2 * cid + hb) * cpb + sid * WCH
      pltpu.sync_copy(psrc_hbm.at[pl.ds(base, WCH)],
                      srcv.at[pl.ds(hb * WCH, WCH)])
      pltpu.sync_copy(pdst_hbm.at[pl.ds(base, WCH)],
                      dstv.at[pl.ds(hb * WCH, WCH)])

    def run(x_hbm, o_hbm, hb):
      @pl.loop(0, rps // ZROWS)
      def _(i):
        pltpu.sync_copy(zbuf, acc.at[pl.ds(row0 + i * ZROWS, ZROWS)])

      plsc.subcore_barrier()

      j0 = hb * WCH
      for b in range(NBW):
        pltpu.async_copy(x_hbm.at[srcv.at[j0 + b]], bufs[b], gsems[b])

      @pl.loop(j0, j0 + WCH, step=NBW)
      def _(j):
        for b in range(NBW):
          pltpu.make_async_copy(x_hbm.at[srcv.at[j + b]], bufs[b],
                                gsems[b]).wait()
          pltpu.async_copy(bufs[b], acc.at[dstv.at[j + b]], ssems[b],
                           add=True)
        for b in range(NBW):
          @pl.when(j + NBW + b < j0 + WCH)
          def _(b=b):
            pltpu.make_async_copy(bufs[b], acc.at[dstv.at[j + b]],
                                  ssems[b]).wait()
            pltpu.async_copy(x_hbm.at[srcv.at[j + NBW + b]], bufs[b],
                             gsems[b])

      for b in range(NBW):
        pltpu.make_async_copy(bufs[b], acc.at[dstv.at[j0 + WCH - NBW + b]],
                              ssems[b]).wait()

      plsc.subcore_barrier()
      pltpu.sync_copy(
          acc.at[pl.ds(row0, rps)],
          o_hbm.at[pl.ds((2 * cid + hb) * QTR + row0, rps)])
      plsc.subcore_barrier()

    for t in range(nt):
      for hb in range(2):
        run(x_hbms[t], o_hbms[t], hb)

  return k(*tables, psrc2d, pdst2d)


def _deg(dst2d):
  cps = (E // (NC * NS)) // CHUNK
  rps = NP // NS

  @functools.partial(
      pl.kernel,
      out_type=(jax.ShapeDtypeStruct((NP, 16), jnp.float32),
                jax.ShapeDtypeStruct((NP, 16), jnp.float32)),
      mesh=plsc.VectorSubcoreMesh(core_axis_name="c", subcore_axis_name="s"),
      compiler_params=pltpu.CompilerParams(use_tc_tiling_on_sc=False),
      scratch_types=[
          pltpu.VMEM((cps, CHUNK), jnp.int32),
          pltpu.VMEM((CHUNK, 16), jnp.float32),
          pltpu.VMEM((ZROWS, 16), jnp.float32),
          pltpu.VMEM_SHARED((NP, 16), jnp.float32),
      ],
  )
  def k(dst_hbm, o0_hbm, o1_hbm, dstv, ones_v, zbuf, acc):
    cid = lax.axis_index("c")
    sid = lax.axis_index("s")

    @pl.loop(0, ZROWS)
    def _(r):
      zbuf[r, pl.ds(0, 16)] = jnp.zeros((16,), jnp.float32)

    @pl.loop(0, CHUNK)
    def _(r):
      ones_v[r, pl.ds(0, 16)] = jnp.ones((16,), jnp.float32)

    row0 = sid * rps

    @pl.loop(0, rps // ZROWS)
    def _(i):
      pltpu.sync_copy(zbuf, acc.at[pl.ds(row0 + i * ZROWS, ZROWS)])

    chunk0 = (cid * NS + sid) * cps
    pltpu.sync_copy(dst_hbm.at[pl.ds(chunk0, cps)], dstv)
    plsc.subcore_barrier()

    @pl.loop(0, cps)
    def _(j):
      pltpu.sync_copy(ones_v, acc.at[dstv.at[j]], add=True)

    plsc.subcore_barrier()
    row_slice = pl.ds(row0, rps)

    @pl.when(cid == 0)
    def _():
      pltpu.sync_copy(acc.at[row_slice], o0_hbm.at[row_slice])

    @pl.when(cid == 1)
    def _():
      pltpu.sync_copy(acc.at[row_slice], o1_hbm.at[row_slice])

  return k(dst2d)


def _post(ps, d0, d1, W, b, relu, widths):
  BN = 2000
  Dout = W.shape[1]
  Din = W.shape[0]
  np_ = len(ps)

  def body(*refs):
    p_refs = refs[:np_]
    d0_ref, d1_ref, w_ref, b_ref = refs[np_:np_ + 4]
    out_refs = refs[np_ + 4:]
    deg = d0_ref[:, 0:1] + d1_ref[:, 0:1]
    inv = 1.0 / jnp.maximum(deg, 1.0)
    h = jnp.concatenate([p[...] * inv for p in p_refs], axis=1)
    y = jnp.dot(h, w_ref[...], preferred_element_type=jnp.float32) + b_ref[...]
    if relu:
      y = jnp.maximum(y, 0.0)
    off = 0
    for r, w in zip(out_refs, widths):
      r[...] = y[:, off:off + w]
      off += w

  grid = (N // BN,)
  return pl.pallas_call(
      body,
      grid=grid,
      in_specs=[pl.BlockSpec((BN, 128), lambda i: (i, 0)) for _ in ps] + [
          pl.BlockSpec((BN, 16), lambda i: (i, 0)),
          pl.BlockSpec((BN, 16), lambda i: (i, 0)),
          pl.BlockSpec((Din, Dout), lambda i: (0, 0)),
          pl.BlockSpec((1, Dout), lambda i: (0, 0)),
      ],
      out_specs=[pl.BlockSpec((BN, w), lambda i: (i, 0)) for w in widths],
      out_shape=[jax.ShapeDtypeStruct((N, w), jnp.float32) for w in widths],
  )(*ps, d0, d1, W, b)


def _blockdiag(Wa, Wb):
  Da, Oa = Wa.shape
  Db, Ob = Wb.shape
  W = jnp.zeros((Da + Db, Oa + Ob), jnp.float32)
  W = W.at[:Da, :Oa].set(Wa)
  W = W.at[Da:, Oa:].set(Wb)
  return W


def kernel(reid_x, st_x, edge_index, reid_W1, reid_b1, reid_W2, reid_b2,
           st_W1, st_b1, st_W2, st_b2, cat_W1, cat_b1, cat_W2, cat_b2):
  src1d = edge_index[0]
  dst1d = edge_index[1]
  dst2d = dst1d.reshape(E // CHUNK, CHUNK)

  ps, pd = _partition(src1d, dst1d)
  psrc2d = ps.reshape(4 * NSLOTS // CHUNK, CHUNK)
  pdst2d = pd.reshape(4 * NSLOTS // CHUNK, CHUNK)

  d0, d1 = _deg(dst2d)

  p1 = _segP((reid_x, st_x), psrc2d, pdst2d)
  Wbd1 = _blockdiag(reid_W1, st_W1)
  bbd1 = jnp.concatenate([reid_b1, st_b1]).reshape(1, -1)
  h1 = _post(p1, d0, d1, Wbd1, bbd1, True, (128, 128))

  p2 = _segP(tuple(h1), psrc2d, pdst2d)
  Wbd2 = _blockdiag(reid_W2, st_W2)
  bbd2 = jnp.concatenate([reid_b2, st_b2]).reshape(1, -1)
  r2 = _post(p2, d0, d1, Wbd2, bbd2, False, (128, 128))

  p3 = _segP(tuple(r2), psrc2d, pdst2d)
  (c1,) = _post(p3, d0, d1, cat_W1, cat_b1.reshape(1, -1), True, (128,))

  p4 = _segP((c1,), psrc2d, pdst2d)
  (out,) = _post(p4, d0, d1, cat_W2, cat_b2.reshape(1, -1), False, (128,))
  return out


# speedup vs baseline: 1.0275x; 1.0275x over previous
import functools

import jax
import jax.numpy as jnp
from jax import lax
from jax.experimental import pallas as pl
from jax.experimental.pallas import tpu as pltpu
from jax.experimental.pallas import tpu_sc as plsc

N = 10000
NP = 10240
E = 320000
NC = 2
NS = 16
DT = 64
CHUNK = 125
ZROWS = 32
NB = 5
ROWS_PER_SUB = NP // NS


def _seg(tables, src2d, dst2d):
  nt = len(tables)
  tpc = nt // NC
  cps = (E // NS) // CHUNK

  @functools.partial(
      pl.kernel,
      out_type=tuple(jax.ShapeDtypeStruct((NP, DT), jnp.float32)
                     for _ in range(nt)),
      mesh=plsc.VectorSubcoreMesh(core_axis_name="c", subcore_axis_name="s"),
      compiler_params=pltpu.CompilerParams(use_tc_tiling_on_sc=False),
      scratch_types=[
          pltpu.VMEM((cps, CHUNK), jnp.int32),
          pltpu.VMEM((cps, CHUNK), jnp.int32),
      ] + [pltpu.VMEM((CHUNK, DT), jnp.float32)] * NB + [
          pltpu.VMEM((ZROWS, DT), jnp.float32),
          pltpu.VMEM_SHARED((NP, DT), jnp.float32),
      ] + [pltpu.SemaphoreType.DMA] * (2 * NB),
  )
  def k(*refs):
    x_hbms = refs[:nt]
    src_hbm, dst_hbm = refs[nt], refs[nt + 1]
    o_hbms = refs[nt + 2:2 * nt + 2]
    rest = refs[2 * nt + 2:]
    srcv, dstv = rest[0], rest[1]
    bufs = rest[2:2 + NB]
    zbuf, acc = rest[2 + NB], rest[3 + NB]
    gsems = rest[4 + NB:4 + 2 * NB]
    ssems = rest[4 + 2 * NB:4 + 3 * NB]

    cid = lax.axis_index("c")
    sid = lax.axis_index("s")
    row0 = sid * ROWS_PER_SUB

    @pl.loop(0, ZROWS)
    def _(r):
      @pl.loop(0, DT, step=16)
      def _(c):
        zbuf[r, pl.ds(c, 16)] = jnp.zeros((16,), jnp.float32)

    pltpu.sync_copy(src_hbm.at[pl.ds(sid * cps, cps)], srcv)
    pltpu.sync_copy(dst_hbm.at[pl.ds(sid * cps, cps)], dstv)

    def run(x_hbm, o_hbm):
      @pl.loop(0, ROWS_PER_SUB // ZROWS)
      def _(i):
        pltpu.sync_copy(zbuf, acc.at[pl.ds(row0 + i * ZROWS, ZROWS)])

      plsc.subcore_barrier()

      for b in range(NB):
        pltpu.async_copy(x_hbm.at[srcv.at[b]], bufs[b], gsems[b])

      @pl.loop(0, cps, step=NB)
      def _(j):
        for b in range(NB):
          pltpu.make_async_copy(x_hbm.at[srcv.at[j + b]], bufs[b],
                                gsems[b]).wait()
          pltpu.async_copy(bufs[b], acc.at[dstv.at[j + b]], ssems[b],
                           add=True)
        for b in range(NB):
          @pl.when(j + NB + b < cps)
          def _(b=b):
            pltpu.make_async_copy(bufs[b], acc.at[dstv.at[j + b]],
                                  ssems[b]).wait()
            pltpu.async_copy(x_hbm.at[srcv.at[j + NB + b]], bufs[b], gsems[b])

      for b in range(NB):
        pltpu.make_async_copy(bufs[b], acc.at[dstv.at[cps - NB + b]],
                              ssems[b]).wait()

      plsc.subcore_barrier()
      pltpu.sync_copy(acc.at[pl.ds(row0, ROWS_PER_SUB)],
                      o_hbm.at[pl.ds(row0, ROWS_PER_SUB)])
      plsc.subcore_barrier()

    for c in range(NC):
      @pl.when(cid == c)
      def _(c=c):
        for t in range(tpc):
          run(x_hbms[c * tpc + t], o_hbms[c * tpc + t])

  return k(*tables, src2d, dst2d)


def _deg(dst2d):
  cps = (E // (NC * NS)) // CHUNK

  @functools.partial(
      pl.kernel,
      out_type=(jax.ShapeDtypeStruct((NP, 16), jnp.float32),
                jax.ShapeDtypeStruct((NP, 16), jnp.float32)),
      mesh=plsc.VectorSubcoreMesh(core_axis_name="c", subcore_axis_name="s"),
      compiler_params=pltpu.CompilerParams(use_tc_tiling_on_sc=False),
      scratch_types=[
          pltpu.VMEM((cps, CHUNK), jnp.int32),
          pltpu.VMEM((CHUNK, 16), jnp.float32),
          pltpu.VMEM((ZROWS, 16), jnp.float32),
          pltpu.VMEM_SHARED((NP, 16), jnp.float32),
      ],
  )
  def k(dst_hbm, o0_hbm, o1_hbm, dstv, ones_v, zbuf, acc):
    cid = lax.axis_index("c")
    sid = lax.axis_index("s")

    @pl.loop(0, ZROWS)
    def _(r):
      zbuf[r, pl.ds(0, 16)] = jnp.zeros((16,), jnp.float32)

    @pl.loop(0, CHUNK)
    def _(r):
      ones_v[r, pl.ds(0, 16)] = jnp.ones((16,), jnp.float32)

    row0 = sid * ROWS_PER_SUB

    @pl.loop(0, ROWS_PER_SUB // ZROWS)
    def _(i):
      pltpu.sync_copy(zbuf, acc.at[pl.ds(row0 + i * ZROWS, ZROWS)])

    chunk0 = (cid * NS + sid) * cps
    pltpu.sync_copy(dst_hbm.at[pl.ds(chunk0, cps)], dstv)
    plsc.subcore_barrier()

    @pl.loop(0, cps)
    def _(j):
      pltpu.sync_copy(ones_v, acc.at[dstv.at[j]], add=True)

    plsc.subcore_barrier()
    row_slice = pl.ds(row0, ROWS_PER_SUB)

    @pl.when(cid == 0)
    def _():
      pltpu.sync_copy(acc.at[row_slice], o0_hbm.at[row_slice])

    @pl.when(cid == 1)
    def _():
      pltpu.sync_copy(acc.at[row_slice], o1_hbm.at[row_slice])

  return k(dst2d)


def _post(ps, d0, d1, W, b, relu, widths):
  BN = 2000
  Dout = W.shape[1]
  Din = W.shape[0]
  np_ = len(ps)

  def body(*refs):
    p_refs = refs[:np_]
    d0_ref, d1_ref, w_ref, b_ref = refs[np_:np_ + 4]
    out_refs = refs[np_ + 4:]
    deg = d0_ref[:, 0:1] + d1_ref[:, 0:1]
    inv = 1.0 / jnp.maximum(deg, 1.0)
    h = jnp.concatenate([p[...] * inv for p in p_refs], axis=1)
    y = jnp.dot(h, w_ref[...], preferred_element_type=jnp.float32) + b_ref[...]
    if relu:
      y = jnp.maximum(y, 0.0)
    off = 0
    for r, w in zip(out_refs, widths):
      r[...] = y[:, off:off + w]
      off += w

  grid = (N // BN,)
  return pl.pallas_call(
      body,
      grid=grid,
      in_specs=[pl.BlockSpec((BN, DT), lambda i: (i, 0)) for _ in ps] + [
          pl.BlockSpec((BN, 16), lambda i: (i, 0)),
          pl.BlockSpec((BN, 16), lambda i: (i, 0)),
          pl.BlockSpec((Din, Dout), lambda i: (0, 0)),
          pl.BlockSpec((1, Dout), lambda i: (0, 0)),
      ],
      out_specs=[pl.BlockSpec((BN, w), lambda i: (i, 0)) for w in widths],
      out_shape=[jax.ShapeDtypeStruct((N, w), jnp.float32) for w in widths],
  )(*ps, d0, d1, W, b)


def _blockdiag(Wa, Wb):
  Da, Oa = Wa.shape
  Db, Ob = Wb.shape
  W = jnp.zeros((Da + Db, Oa + Ob), jnp.float32)
  W = W.at[:Da, :Oa].set(Wa)
  W = W.at[Da:, Oa:].set(Wb)
  return W


def kernel(reid_x, st_x, edge_index, reid_W1, reid_b1, reid_W2, reid_b2,
           st_W1, st_b1, st_W2, st_b2, cat_W1, cat_b1, cat_W2, cat_b2):
  src2d = edge_index[0].reshape(E // CHUNK, CHUNK)
  dst2d = edge_index[1].reshape(E // CHUNK, CHUNK)

  d0, d1 = _deg(dst2d)

  x_tables = (reid_x[:, :DT], reid_x[:, DT:], st_x[:, :DT], st_x[:, DT:])
  p1 = _seg(x_tables, src2d, dst2d)
  Wbd1 = _blockdiag(reid_W1, st_W1)
  bbd1 = jnp.concatenate([reid_b1, st_b1]).reshape(1, -1)
  h1 = _post(p1, d0, d1, Wbd1, bbd1, True, (DT,) * 4)

  p2 = _seg(tuple(h1), src2d, dst2d)
  Wbd2 = _blockdiag(reid_W2, st_W2)
  bbd2 = jnp.concatenate([reid_b2, st_b2]).reshape(1, -1)
  r2 = _post(p2, d0, d1, Wbd2, bbd2, False, (DT,) * 4)

  p3 = _seg(tuple(r2), src2d, dst2d)
  c1 = _post(p3, d0, d1, cat_W1, cat_b1.reshape(1, -1), True, (DT, DT))

  p4 = _seg(tuple(c1), src2d, dst2d)
  (out,) = _post(p4, d0, d1, cat_W2, cat_b2.reshape(1, -1), False, (128,))
  return out
